# baseline (speedup 1.0000x reference)
import jax
import jax.numpy as jnp
from jax.experimental import pallas as pl
from jax.experimental.pallas import tpu as pltpu

_PACK = 4
_ROWS = 65536
_D = 32
_PROWS = _ROWS // _PACK
_PD = _D * _PACK
_BLK = 2048


def _matmul_block(x_ref, w_ref, b_ref, o_ref):
    o_ref[...] = (
        jax.lax.dot_general(
            x_ref[...], w_ref[...],
            dimension_numbers=(((1,), (0,)), ((), ())),
            preferred_element_type=jnp.float32,
        )
        + b_ref[...]
    )


def kernel(x, W, b):
    x4 = x.reshape(_PROWS, _PD)
    eye = jnp.eye(_PACK, dtype=W.dtype)
    wbd = jnp.einsum("pq,io->piqo", eye, W).reshape(_PD, _PD)
    b4 = jnp.tile(b, _PACK).reshape(1, _PD)

    y4 = pl.pallas_call(
        _matmul_block,
        grid=(_PROWS // _BLK,),
        in_specs=[
            pl.BlockSpec((_BLK, _PD), lambda i: (i, 0)),
            pl.BlockSpec((_PD, _PD), lambda i: (0, 0)),
            pl.BlockSpec((1, _PD), lambda i: (0, 0)),
        ],
        out_specs=pl.BlockSpec((_BLK, _PD), lambda i: (i, 0)),
        out_shape=jax.ShapeDtypeStruct((_PROWS, _PD), jnp.float32),
        compiler_params=pltpu.CompilerParams(
            dimension_semantics=("arbitrary",),
        ),
    )(x4, wbd, b4)

    return y4.reshape(_ROWS, _D)


# speedup vs baseline: 1.2449x; 1.2449x over previous
import jax
import jax.numpy as jnp
from jax.experimental import pallas as pl
from jax.experimental.pallas import tpu as pltpu

_ROWS = 65536
_D = 32
_BLK = 2048


def _matmul_block(x_ref, w_ref, b_ref, o_ref):
    o_ref[...] = (
        jax.lax.dot_general(
            x_ref[...], w_ref[...],
            dimension_numbers=(((1,), (0,)), ((), ())),
            preferred_element_type=jnp.float32,
        )
        + b_ref[...]
    )


def kernel(x, W, b):
    return pl.pallas_call(
        _matmul_block,
        grid=(_ROWS // _BLK,),
        in_specs=[
            pl.BlockSpec((_BLK, _D), lambda i: (i, 0)),
            pl.BlockSpec((_D, _D), lambda i: (0, 0)),
            pl.BlockSpec((_D,), lambda i: (0,)),
        ],
        out_specs=pl.BlockSpec((_BLK, _D), lambda i: (i, 0)),
        out_shape=jax.ShapeDtypeStruct((_ROWS, _D), jnp.float32),
        compiler_params=pltpu.CompilerParams(
            dimension_semantics=("parallel",),
        ),
    )(x, W, b)


# speedup vs baseline: 1.5045x; 1.2086x over previous
import jax
import jax.numpy as jnp
from jax.experimental import pallas as pl
from jax.experimental.pallas import tpu as pltpu

_ROWS = 65536
_D = 32
_BLK = 8192


def _matmul_block(x_ref, w_ref, b_ref, o_ref):
    o_ref[...] = (
        jax.lax.dot_general(
            x_ref[...], w_ref[...],
            dimension_numbers=(((1,), (0,)), ((), ())),
            preferred_element_type=jnp.float32,
        )
        + b_ref[...]
    )


def kernel(x, W, b):
    return pl.pallas_call(
        _matmul_block,
        grid=(_ROWS // _BLK,),
        in_specs=[
            pl.BlockSpec((_BLK, _D), lambda i: (i, 0)),
            pl.BlockSpec((_D, _D), lambda i: (0, 0)),
            pl.BlockSpec((_D,), lambda i: (0,)),
        ],
        out_specs=pl.BlockSpec((_BLK, _D), lambda i: (i, 0)),
        out_shape=jax.ShapeDtypeStruct((_ROWS, _D), jnp.float32),
        compiler_params=pltpu.CompilerParams(
            dimension_semantics=("parallel",),
        ),
    )(x, W, b)


# speedup vs baseline: 9.2503x; 6.1484x over previous
import jax
import jax.numpy as jnp
from jax.experimental import pallas as pl
from jax.experimental.pallas import tpu as pltpu

_ROWS = 65536
_D = 32
_CBLK = 8192


def _matmul_t_block(xt_ref, w_ref, b_ref, o_ref):
    yt = jax.lax.dot_general(
        w_ref[...], xt_ref[...],
        dimension_numbers=(((0,), (0,)), ((), ())),
        preferred_element_type=jnp.float32,
    )
    o_ref[...] = yt + jax.lax.broadcast_in_dim(b_ref[...], (_D, _CBLK), (0,))


def kernel(x, W, b):
    xt = x.T
    yt = pl.pallas_call(
        _matmul_t_block,
        grid=(_ROWS // _CBLK,),
        in_specs=[
            pl.BlockSpec((_D, _CBLK), lambda i: (0, i)),
            pl.BlockSpec((_D, _D), lambda i: (0, 0)),
            pl.BlockSpec((_D,), lambda i: (0,)),
        ],
        out_specs=pl.BlockSpec((_D, _CBLK), lambda i: (0, i)),
        out_shape=jax.ShapeDtypeStruct((_D, _ROWS), jnp.float32),
        compiler_params=pltpu.CompilerParams(
            dimension_semantics=("parallel",),
        ),
    )(xt, W, b)
    return yt.T
